# Initial kernel scaffold; baseline (speedup 1.0000x reference)
import jax
import jax.numpy as jnp
from jax.experimental import pallas as pl


def kernel(x, table):
    raise NotImplementedError("write your pallas kernel here")


# speedup vs baseline: 1.6927x; 1.6927x over previous
import functools

import jax
import jax.numpy as jnp
from jax import lax
from jax.experimental import pallas as pl
from jax.experimental.pallas import tpu as pltpu
from jax.experimental.pallas import tpu_sc as plsc

EMBED_DIM = 64
N_TOKENS = 16384 * 50

_info = plsc.get_sparse_core_info()
_NC = _info.num_cores
_NS = _info.num_subcores
_NW = _NC * _NS
_B_PER_W = N_TOKENS // _NW
_CHUNK = 128
_NCHUNK = _B_PER_W // _CHUNK

_mesh = plsc.VectorSubcoreMesh(core_axis_name="c", subcore_axis_name="s")


@functools.partial(
    pl.kernel,
    mesh=_mesh,
    out_type=jax.ShapeDtypeStruct((N_TOKENS, EMBED_DIM), jnp.float32),
    scratch_types=[
        pltpu.VMEM((_NCHUNK, _CHUNK), jnp.int32),
        pltpu.VMEM((_CHUNK, EMBED_DIM), jnp.float32),
        pltpu.SemaphoreType.DMA,
    ],
    compiler_params=pltpu.CompilerParams(use_tc_tiling_on_sc=False),
)
def _sc_gather(idx_hbm, table_hbm, out_hbm, idx_v, rows_v, sem):
    wid = lax.axis_index("s") * _NC + lax.axis_index("c")
    base = wid * _B_PER_W
    pltpu.sync_copy(idx_hbm.at[pl.ds(wid * _NCHUNK, _NCHUNK)], idx_v)

    def chunk_body(g, _):
        pltpu.async_copy(table_hbm.at[idx_v.at[g]], rows_v, sem).wait()
        pltpu.sync_copy(rows_v, out_hbm.at[pl.ds(base + g * _CHUNK, _CHUNK)])
        return ()

    lax.fori_loop(0, _NCHUNK, chunk_body, (), unroll=False)


def kernel(x, table):
    idx = jnp.reshape(x.astype(jnp.int32), (N_TOKENS // _CHUNK, _CHUNK))
    out = _sc_gather(idx, table)
    return jnp.reshape(out, (*x.shape, EMBED_DIM))


# speedup vs baseline: 1.8768x; 1.1088x over previous
import functools

import jax
import jax.numpy as jnp
from jax import lax
from jax.experimental import pallas as pl
from jax.experimental.pallas import tpu as pltpu
from jax.experimental.pallas import tpu_sc as plsc

EMBED_DIM = 64
N_TOKENS = 16384 * 50

_info = plsc.get_sparse_core_info()
_NC = _info.num_cores
_NS = _info.num_subcores
_NW = _NC * _NS
_B_PER_W = N_TOKENS // _NW
_CHUNK = 128
_NCHUNK = _B_PER_W // _CHUNK
_K = 8
_P = 4
_NG = _NCHUNK // _K

_mesh = plsc.VectorSubcoreMesh(core_axis_name="c", subcore_axis_name="s")


@functools.partial(
    pl.kernel,
    mesh=_mesh,
    out_type=jax.ShapeDtypeStruct((N_TOKENS, EMBED_DIM), jnp.float32),
    scratch_types=[
        pltpu.VMEM((_NCHUNK, _CHUNK), jnp.int32),
        pltpu.VMEM((_K, _CHUNK, EMBED_DIM), jnp.float32),
    ]
    + [pltpu.SemaphoreType.DMA] * (2 * _K),
    compiler_params=pltpu.CompilerParams(use_tc_tiling_on_sc=False),
)
def _sc_gather(idx_hbm, table_hbm, out_hbm, idx_v, rows_v, *sems):
    gsem = sems[:_K]
    osem = sems[_K:]
    wid = lax.axis_index("s") * _NC + lax.axis_index("c")
    base = wid * _B_PER_W
    pltpu.sync_copy(idx_hbm.at[pl.ds(wid * _NCHUNK, _NCHUNK)], idx_v)

    def gather_start(g, b):
        pltpu.async_copy(table_hbm.at[idx_v.at[g]], rows_v.at[b], gsem[b])

    def gather_wait(g, b):
        pltpu.make_async_copy(table_hbm.at[idx_v.at[g]], rows_v.at[b], gsem[b]).wait()

    def out_start(g, b):
        pltpu.async_copy(rows_v.at[b], out_hbm.at[pl.ds(base + g * _CHUNK, _CHUNK)], osem[b])

    def out_wait(g, b):
        pltpu.make_async_copy(rows_v.at[b], out_hbm.at[pl.ds(base + g * _CHUNK, _CHUNK)], osem[b]).wait()

    for c in range(_P):
        gather_start(c, c)

    for b in range(_K):
        if b + _P - _K >= 0:
            out_wait(b + _P - _K, (b + _P) % _K)
        gather_start(b + _P, (b + _P) % _K)
        gather_wait(b, b)
        out_start(b, b)

    def group_body(i, _):
        for b in range(_K):
            g = i * _K + b
            out_wait(g + _P - _K, (b + _P) % _K)
            gather_start(g + _P, (b + _P) % _K)
            gather_wait(g, b)
            out_start(g, b)
        return ()

    lax.fori_loop(1, _NG - 1, group_body, (), unroll=False)

    for b in range(_K):
        g = (_NG - 1) * _K + b
        out_wait(g + _P - _K, (b + _P) % _K)
        if b < _K - _P:
            gather_start(g + _P, (b + _P) % _K)
        gather_wait(g, b)
        out_start(g, b)

    for j in range(_K - _P):
        c = _NCHUNK - (_K - _P) + j
        out_wait(c, c % _K)


def kernel(x, table):
    idx = jnp.reshape(x.astype(jnp.int32), (N_TOKENS // _CHUNK, _CHUNK))
    out = _sc_gather(idx, table)
    return jnp.reshape(out, (*x.shape, EMBED_DIM))
